# baseline (speedup 1.0000x reference)
import functools
import math

import jax
import jax.numpy as jnp
from jax import lax
from jax.experimental import pallas as pl
from jax.experimental.pallas import tpu as pltpu
from jax.experimental.pallas import tpu_sc as plsc

f32 = jnp.float32
NI = 4
DH = 16
DHID = 64
D1 = 16
D2 = 8
SC = 0.125

NC, NS = 2, 16
NW = NC * NS
CH = 128


def _pad_len(n):
  per_w = -(-n // NW)
  per_w = -(-per_w // CH) * CH
  return per_w * NW, per_w


def _sc_gather(n_pad, per_w, d):
  nch = per_w // CH
  mesh = plsc.VectorSubcoreMesh(core_axis_name="c", subcore_axis_name="s")

  @functools.partial(
      pl.kernel, mesh=mesh,
      out_type=jax.ShapeDtypeStruct((n_pad, d), f32),
      scratch_types=[
          pltpu.VMEM((CH,), jnp.int32), pltpu.VMEM((CH,), jnp.int32),
          pltpu.VMEM((CH, d), f32), pltpu.VMEM((CH, d), f32),
          pltpu.SemaphoreType.DMA, pltpu.SemaphoreType.DMA,
          pltpu.SemaphoreType.DMA, pltpu.SemaphoreType.DMA,
      ],
  )
  def gat(idx_hbm, table_hbm, out_hbm, idx0, idx1, rows0, rows1,
          g0, g1, s0, s1):
    wid = lax.axis_index("s") * NC + lax.axis_index("c")
    base = wid * per_w
    idx_v = [idx0, idx1]
    rows_v = [rows0, rows1]
    gsem = [g0, g1]
    ssem = [s0, s1]
    gcopy = [None, None]
    scopy = [None, None]
    for c in range(nch + 1):
      if c < nch:
        b = c % 2
        if scopy[b] is not None:
          scopy[b].wait()
        off = base + c * CH
        pltpu.sync_copy(idx_hbm.at[pl.ds(off, CH)], idx_v[b])
        gcopy[b] = pltpu.make_async_copy(
            table_hbm.at[idx_v[b]], rows_v[b], gsem[b])
        gcopy[b].start()
      if c >= 1:
        pb = (c - 1) % 2
        gcopy[pb].wait()
        off = base + (c - 1) * CH
        scopy[pb] = pltpu.make_async_copy(
            rows_v[pb], out_hbm.at[pl.ds(off, CH)], ssem[pb])
        scopy[pb].start()
    scopy[(nch - 1) % 2].wait()
    if nch >= 2:
      scopy[nch % 2].wait()

  return gat


BB = 200
RP = BB * D1 // 2
G = D1 // 2


def _iota2(shape, d):
  return lax.broadcasted_iota(jnp.int32, shape, d)


def _halfsum(n):
  return (_iota2((2 * n, n), 0) % n == _iota2((2 * n, n), 1)).astype(f32)


def _dupm(n):
  return (_iota2((n, 2 * n), 0) == _iota2((n, 2 * n), 1) % n).astype(f32)


def _bdiag(w):
  z = jnp.zeros_like(w)
  top = jnp.concatenate([w, z], axis=1)
  bot = jnp.concatenate([z, w], axis=1)
  return jnp.concatenate([top, bot], axis=0)


def _masks():
  i16r = _iota2((16, 64), 0)
  i64c = _iota2((16, 64), 1)
  tq = [(((i16r // 4) == q) & ((i16r % 4) == (i64c // DH))).astype(f32)
        for q in range(NI)]
  jm = ((_iota2((16, 16), 0) % 4) == (_iota2((16, 16), 1) % 4)).astype(f32)
  i64r = _iota2((64, 16), 0)
  i16c = _iota2((64, 16), 1)
  segj = [(((i16c // 4) == j) & ((i16c % 4) == (i64r // DH))).astype(f32)
          for j in range(NI)]
  eq = ((_iota2((4, 16), 1) // 4) == _iota2((4, 16), 0)).astype(f32)
  sege = ((i64r // DH) == (i16c % 4)).astype(f32)
  mq = [((_iota2((16, 64), 0) == q * 4 + _iota2((16, 64), 1) // DH)
         ).astype(f32) for q in range(NI)]
  return tq, jm, segj, eq, sege, mq


def _gsum(x, n, g):
  return jnp.sum(x.reshape(n, g, x.shape[-1]), axis=1)


def _gexp(x, n, g):
  return jnp.broadcast_to(x[:, None, :], (n, g, x.shape[-1])).reshape(
      n * g, x.shape[-1])


def _isab_body(mbx, i0, wq0, bq0, i0t, wq0t, bq0c,
               wk0b, bk0d, wv0b, bv0d, wo0, bo0,
               wq1b, bq1d, wk1, bk1, wv1, bv1, wo1b, bo1d,
               wd, bd, pad128, out_e):
  dot = functools.partial(jnp.dot, preferred_element_type=f32)
  tq, jm, segj, eq, sege, mq = _masks()
  jmb = _bdiag(jm)
  h64 = _halfsum(64)
  h16 = _halfsum(16)
  d64 = _dupm(64)

  v2 = mbx[...]
  kp0 = dot(v2, wk0b[...]) + bk0d[...]
  vp0 = dot(v2, wv0b[...]) + bv0d[...]

  qp0 = dot(i0[...], wq0[...]) + bq0[...]
  qp0t = dot(wq0t[...], i0t[...]) + bq0c[...]
  qt = dot(qp0t, eq) * sege
  qtb = _bdiag(qt)

  s = dot(kp0, qtb)
  e = jnp.exp(s * SC)
  den = _gsum(e, BB, G)
  dinv = 1.0 / dot(den, h16)

  hs = []
  for q in range(NI):
    uq = dot(e, _bdiag(tq[q]))
    gs = _gsum(uq * vp0, BB, G)
    oq = dot(gs, h64) * dot(dinv, mq[q]) + qp0[q:q + 1, :]
    hs.append(oq + jax.nn.relu(dot(oq, wo0[...]) + bo0[...]))

  qp1 = dot(v2, wq1b[...]) + bq1d[...]
  s1 = None
  vjs = []
  for j in range(NI):
    kj = dot(hs[j], wk1[...]) + bk1[...]
    vjs.append(dot(hs[j], wv1[...]) + bv1[...])
    kje = _gexp(dot(kj, d64), BB, G)
    t = dot(qp1 * kje, _bdiag(segj[j]))
    s1 = t if s1 is None else s1 + t
  e1 = jnp.exp(s1 * SC)
  den1 = dot(e1, jmb)
  a1 = e1 / den1
  acc = None
  for j in range(NI):
    a1j = dot(a1, _bdiag(tq[j]))
    t = a1j * _gexp(dot(vjs[j], d64), BB, G)
    acc = t if acc is None else acc + t
  o1 = qp1 + acc
  vp = o1 + jax.nn.relu(dot(o1, wo1b[...]) + bo1d[...])

  em = dot(_gsum(vp, BB, G), h64) * (1.0 / D1)
  ef = dot(em, wd[...]) + bd[...]
  out_e[...] = dot(ef, pad128[...])


def _isab_call(mailbox2, wts, n_hedges):
  grid = n_hedges // BB
  full = lambda a: pl.BlockSpec(a.shape, lambda i: (0,) * a.ndim)
  in_specs = [pl.BlockSpec((RP, 256), lambda i: (i, 0))]
  in_specs += [full(w) for w in wts]
  return pl.pallas_call(
      _isab_body,
      grid=(grid,),
      in_specs=in_specs,
      out_specs=pl.BlockSpec((BB, 128), lambda i: (i, 0)),
      out_shape=jax.ShapeDtypeStruct((n_hedges, 128), f32),
  )(mailbox2, *wts)


def _bd(w):
  z = jnp.zeros_like(w)
  return jnp.block([[w, z], [z, w]])


def _isab_weights(p):
  b = lambda x: x.reshape(1, -1)
  d = lambda x: b(jnp.concatenate([x, x]))
  m0, m1 = p['mab0'], p['mab1']
  pad128 = jnp.concatenate(
      [jnp.eye(DHID, dtype=f32), jnp.zeros((DHID, 64), f32)], axis=1)
  return [p['I'][0], m0['Wq'], b(m0['bq']),
          p['I'][0].T, m0['Wq'].T, m0['bq'].reshape(-1, 1),
          _bd(m0['Wk']), d(m0['bk']), _bd(m0['Wv']), d(m0['bv']),
          m0['Wo'], b(m0['bo']),
          _bd(m1['Wq']), d(m1['bq']), m1['Wk'], b(m1['bk']),
          m1['Wv'], b(m1['bv']), _bd(m1['Wo']), d(m1['bo']),
          p['Wd'], b(p['bd']), pad128]


BN = 400
GA = D2 // 2
RPA = BN * GA


def _agg_body(vf, eg8, wqv, bqv, wket8, bkec, wvep, bve, out):
  dot = functools.partial(jnp.dot, preferred_element_type=f32)
  ones8 = (_iota2((1024, D2), 0) // 128 == _iota2((1024, D2), 1)).astype(f32)
  sel8 = (_iota2((D2, 1024), 0) == _iota2((D2, 1024), 1) // 128).astype(f32)
  sum8 = (_iota2((1024, 128), 0) % 128 == _iota2((1024, 128), 1)).astype(f32)

  q = dot(vf[...], wqv[...]) + bqv[...]
  qtp8 = dot(q, wket8[...])
  sb = dot(q, bkec[...])
  eg = eg8[...]
  s = dot(eg * qtp8, ones8) + sb
  s = jnp.where(s >= 0, s, 0.01 * s) * SC
  m = jnp.max(s, axis=1, keepdims=True)
  a = jnp.exp(s - m)
  a = a / jnp.sum(a, axis=1, keepdims=True)
  z = dot(eg * dot(a, sel8), sum8)
  out[...] = jax.nn.relu(dot(z, wvep[...]) + bve[...])


def _agg_call(vfeat, eg8, wts, n_nodes):
  grid = n_nodes // BN
  full = lambda a: pl.BlockSpec(a.shape, lambda i: (0,) * a.ndim)
  return pl.pallas_call(
      _agg_body,
      grid=(grid,),
      in_specs=[
          pl.BlockSpec((BN, 128), lambda i: (i, 0)),
          pl.BlockSpec((BN, 1024), lambda i: (i, 0)),
      ] + [full(w) for w in wts],
      out_specs=pl.BlockSpec((BN, 128), lambda i: (i, 0)),
      out_shape=jax.ShapeDtypeStruct((n_nodes, 128), f32),
  )(vfeat, eg8, *wts)


def _agg_weights(p):
  b = lambda x: x.reshape(1, -1)
  wketp = jnp.concatenate([p['Wke'].T, jnp.zeros((DHID, 64), f32)], axis=1)
  wket8 = jnp.concatenate([wketp] * D2, axis=1)
  bkec = p['bke'].reshape(-1, 1)
  wvep = jnp.concatenate([p['Wve'], jnp.zeros((64, 128), f32)], axis=0)
  return [p['Wqv'], b(p['bqv']), wket8, bkec, wvep, b(p['bve'])]


def kernel(vfeat, efeat, params, nbr_nodes, nbr_edges):
  n_nodes, din_v = vfeat.shape
  n_hedges = nbr_nodes.shape[0]
  dout_e = params['Wd'].shape[1]

  idx1 = nbr_nodes.reshape(-1)
  n1_pad, per_w1 = _pad_len(idx1.shape[0])
  idx1 = jnp.concatenate(
      [idx1, jnp.zeros((n1_pad - idx1.shape[0],), jnp.int32)])
  mailbox = _sc_gather(n1_pad, per_w1, din_v)(idx1, vfeat)

  etab = _isab_call(mailbox.reshape(-1, 2 * din_v), _isab_weights(params),
                    n_hedges)
  efeat_new = etab[:, :dout_e]

  idx2 = nbr_edges.reshape(-1)
  n2_pad, per_w2 = _pad_len(idx2.shape[0])
  idx2 = jnp.concatenate(
      [idx2, jnp.zeros((n2_pad - idx2.shape[0],), jnp.int32)])
  eg = _sc_gather(n2_pad, per_w2, 128)(idx2, etab)

  vfeat_new = _agg_call(vfeat, eg.reshape(-1, D2 * 128), _agg_weights(params),
                        n_nodes)
  return (vfeat_new, efeat_new)


# speedup vs baseline: 1.0203x; 1.0203x over previous
import functools
import math

import jax
import jax.numpy as jnp
from jax import lax
from jax.experimental import pallas as pl
from jax.experimental.pallas import tpu as pltpu
from jax.experimental.pallas import tpu_sc as plsc

f32 = jnp.float32
NI = 4
DH = 16
DHID = 64
D1 = 16
D2 = 8
SC = 0.125

NC, NS = 2, 16
NW = NC * NS
CH = 128


def _pad_len(n):
  per_w = -(-n // NW)
  per_w = -(-per_w // CH) * CH
  return per_w * NW, per_w


NBUF = 4


def _sc_gather(n_pad, per_w, d):
  nch = per_w // CH
  mesh = plsc.VectorSubcoreMesh(core_axis_name="c", subcore_axis_name="s")

  @functools.partial(
      pl.kernel, mesh=mesh,
      out_type=jax.ShapeDtypeStruct((n_pad, d), f32),
      scratch_types=[pltpu.VMEM((per_w,), jnp.int32)]
      + [pltpu.VMEM((CH, d), f32) for _ in range(NBUF)]
      + [pltpu.SemaphoreType.DMA for _ in range(2 * NBUF)],
  )
  def gat(idx_hbm, table_hbm, out_hbm, idx_v, *bufs_sems):
    rows_v = list(bufs_sems[:NBUF])
    gsem = list(bufs_sems[NBUF:2 * NBUF])
    ssem = list(bufs_sems[2 * NBUF:])
    wid = lax.axis_index("s") * NC + lax.axis_index("c")
    base = wid * per_w
    pltpu.sync_copy(idx_hbm.at[pl.ds(base, per_w)], idx_v)
    gcopy = [None] * NBUF
    scopy = [None] * NBUF
    lag = 2
    for c in range(nch + lag):
      if c < nch:
        b = c % NBUF
        if scopy[b] is not None:
          scopy[b].wait()
        gcopy[b] = pltpu.make_async_copy(
            table_hbm.at[idx_v.at[pl.ds(c * CH, CH)]], rows_v[b], gsem[b])
        gcopy[b].start()
      if c >= lag:
        pb = (c - lag) % NBUF
        gcopy[pb].wait()
        scopy[pb] = pltpu.make_async_copy(
            rows_v[pb], out_hbm.at[pl.ds(base + (c - lag) * CH, CH)],
            ssem[pb])
        scopy[pb].start()
    for k in range(min(NBUF, nch)):
      scopy[(nch - 1 - k) % NBUF].wait()

  return gat


BB = 200
RP = BB * D1 // 2
G = D1 // 2


def _iota2(shape, d):
  return lax.broadcasted_iota(jnp.int32, shape, d)


def _halfsum(n):
  return (_iota2((2 * n, n), 0) % n == _iota2((2 * n, n), 1)).astype(f32)


def _dupm(n):
  return (_iota2((n, 2 * n), 0) == _iota2((n, 2 * n), 1) % n).astype(f32)


def _bdiag(w):
  z = jnp.zeros_like(w)
  top = jnp.concatenate([w, z], axis=1)
  bot = jnp.concatenate([z, w], axis=1)
  return jnp.concatenate([top, bot], axis=0)


def _masks():
  i16r = _iota2((16, 64), 0)
  i64c = _iota2((16, 64), 1)
  tq = [(((i16r // 4) == q) & ((i16r % 4) == (i64c // DH))).astype(f32)
        for q in range(NI)]
  jm = ((_iota2((16, 16), 0) % 4) == (_iota2((16, 16), 1) % 4)).astype(f32)
  i64r = _iota2((64, 16), 0)
  i16c = _iota2((64, 16), 1)
  segj = [(((i16c // 4) == j) & ((i16c % 4) == (i64r // DH))).astype(f32)
          for j in range(NI)]
  eq = ((_iota2((4, 16), 1) // 4) == _iota2((4, 16), 0)).astype(f32)
  sege = ((i64r // DH) == (i16c % 4)).astype(f32)
  mq = [((_iota2((16, 64), 0) == q * 4 + _iota2((16, 64), 1) // DH)
         ).astype(f32) for q in range(NI)]
  return tq, jm, segj, eq, sege, mq


def _gsum(x, n, g):
  return jnp.sum(x.reshape(n, g, x.shape[-1]), axis=1)


def _gexp(x, n, g):
  return jnp.broadcast_to(x[:, None, :], (n, g, x.shape[-1])).reshape(
      n * g, x.shape[-1])


def _isab_body(mbx, i0, wq0, bq0, i0t, wq0t, bq0c,
               wk0b, bk0d, wv0b, bv0d, wo0, bo0,
               wq1b, bq1d, wk1, bk1, wv1, bv1, wo1b, bo1d,
               wd, bd, pad128, out_e):
  dot = functools.partial(jnp.dot, preferred_element_type=f32)
  tq, jm, segj, eq, sege, mq = _masks()
  jmb = _bdiag(jm)
  h64 = _halfsum(64)
  h16 = _halfsum(16)
  d64 = _dupm(64)

  v2 = mbx[...]
  kp0 = dot(v2, wk0b[...]) + bk0d[...]
  vp0 = dot(v2, wv0b[...]) + bv0d[...]

  qp0 = dot(i0[...], wq0[...]) + bq0[...]
  qp0t = dot(wq0t[...], i0t[...]) + bq0c[...]
  qt = dot(qp0t, eq) * sege
  qtb = _bdiag(qt)

  s = dot(kp0, qtb)
  e = jnp.exp(s * SC)
  den = _gsum(e, BB, G)
  dinv = 1.0 / dot(den, h16)

  hs = []
  for q in range(NI):
    uq = dot(e, _bdiag(tq[q]))
    gs = _gsum(uq * vp0, BB, G)
    oq = dot(gs, h64) * dot(dinv, mq[q]) + qp0[q:q + 1, :]
    hs.append(oq + jax.nn.relu(dot(oq, wo0[...]) + bo0[...]))

  qp1 = dot(v2, wq1b[...]) + bq1d[...]
  s1 = None
  vjs = []
  for j in range(NI):
    kj = dot(hs[j], wk1[...]) + bk1[...]
    vjs.append(dot(hs[j], wv1[...]) + bv1[...])
    kje = _gexp(dot(kj, d64), BB, G)
    t = dot(qp1 * kje, _bdiag(segj[j]))
    s1 = t if s1 is None else s1 + t
  e1 = jnp.exp(s1 * SC)
  den1 = dot(e1, jmb)
  a1 = e1 / den1
  acc = None
  for j in range(NI):
    a1j = dot(a1, _bdiag(tq[j]))
    t = a1j * _gexp(dot(vjs[j], d64), BB, G)
    acc = t if acc is None else acc + t
  o1 = qp1 + acc
  vp = o1 + jax.nn.relu(dot(o1, wo1b[...]) + bo1d[...])

  em = dot(_gsum(vp, BB, G), h64) * (1.0 / D1)
  ef = dot(em, wd[...]) + bd[...]
  out_e[...] = dot(ef, pad128[...])


def _isab_call(mailbox2, wts, n_hedges):
  grid = n_hedges // BB
  full = lambda a: pl.BlockSpec(a.shape, lambda i: (0,) * a.ndim)
  in_specs = [pl.BlockSpec((RP, 256), lambda i: (i, 0))]
  in_specs += [full(w) for w in wts]
  return pl.pallas_call(
      _isab_body,
      grid=(grid,),
      in_specs=in_specs,
      out_specs=pl.BlockSpec((BB, 128), lambda i: (i, 0)),
      out_shape=jax.ShapeDtypeStruct((n_hedges, 128), f32),
  )(mailbox2, *wts)


def _bd(w):
  z = jnp.zeros_like(w)
  return jnp.block([[w, z], [z, w]])


def _isab_weights(p):
  b = lambda x: x.reshape(1, -1)
  d = lambda x: b(jnp.concatenate([x, x]))
  m0, m1 = p['mab0'], p['mab1']
  pad128 = jnp.concatenate(
      [jnp.eye(DHID, dtype=f32), jnp.zeros((DHID, 64), f32)], axis=1)
  return [p['I'][0], m0['Wq'], b(m0['bq']),
          p['I'][0].T, m0['Wq'].T, m0['bq'].reshape(-1, 1),
          _bd(m0['Wk']), d(m0['bk']), _bd(m0['Wv']), d(m0['bv']),
          m0['Wo'], b(m0['bo']),
          _bd(m1['Wq']), d(m1['bq']), m1['Wk'], b(m1['bk']),
          m1['Wv'], b(m1['bv']), _bd(m1['Wo']), d(m1['bo']),
          p['Wd'], b(p['bd']), pad128]


BN = 400
GA = D2 // 2
RPA = BN * GA


def _agg_body(vf, eg8, wqv, bqv, wket8, bkec, wvep, bve, out):
  dot = functools.partial(jnp.dot, preferred_element_type=f32)
  ones8 = (_iota2((1024, D2), 0) // 128 == _iota2((1024, D2), 1)).astype(f32)
  sel8 = (_iota2((D2, 1024), 0) == _iota2((D2, 1024), 1) // 128).astype(f32)
  sum8 = (_iota2((1024, 128), 0) % 128 == _iota2((1024, 128), 1)).astype(f32)

  q = dot(vf[...], wqv[...]) + bqv[...]
  qtp8 = dot(q, wket8[...])
  sb = dot(q, bkec[...])
  eg = eg8[...]
  s = dot(eg * qtp8, ones8) + sb
  s = jnp.where(s >= 0, s, 0.01 * s) * SC
  m = jnp.max(s, axis=1, keepdims=True)
  a = jnp.exp(s - m)
  a = a / jnp.sum(a, axis=1, keepdims=True)
  z = dot(eg * dot(a, sel8), sum8)
  out[...] = jax.nn.relu(dot(z, wvep[...]) + bve[...])


def _agg_call(vfeat, eg8, wts, n_nodes):
  grid = n_nodes // BN
  full = lambda a: pl.BlockSpec(a.shape, lambda i: (0,) * a.ndim)
  return pl.pallas_call(
      _agg_body,
      grid=(grid,),
      in_specs=[
          pl.BlockSpec((BN, 128), lambda i: (i, 0)),
          pl.BlockSpec((BN, 1024), lambda i: (i, 0)),
      ] + [full(w) for w in wts],
      out_specs=pl.BlockSpec((BN, 128), lambda i: (i, 0)),
      out_shape=jax.ShapeDtypeStruct((n_nodes, 128), f32),
  )(vfeat, eg8, *wts)


def _agg_weights(p):
  b = lambda x: x.reshape(1, -1)
  wketp = jnp.concatenate([p['Wke'].T, jnp.zeros((DHID, 64), f32)], axis=1)
  wket8 = jnp.concatenate([wketp] * D2, axis=1)
  bkec = p['bke'].reshape(-1, 1)
  wvep = jnp.concatenate([p['Wve'], jnp.zeros((64, 128), f32)], axis=0)
  return [p['Wqv'], b(p['bqv']), wket8, bkec, wvep, b(p['bve'])]


def kernel(vfeat, efeat, params, nbr_nodes, nbr_edges):
  n_nodes, din_v = vfeat.shape
  n_hedges = nbr_nodes.shape[0]
  dout_e = params['Wd'].shape[1]

  idx1 = nbr_nodes.reshape(-1)
  n1_pad, per_w1 = _pad_len(idx1.shape[0])
  idx1 = jnp.concatenate(
      [idx1, jnp.zeros((n1_pad - idx1.shape[0],), jnp.int32)])
  mailbox = _sc_gather(n1_pad, per_w1, din_v)(idx1, vfeat)

  etab = _isab_call(mailbox.reshape(-1, 2 * din_v), _isab_weights(params),
                    n_hedges)
  efeat_new = etab[:, :dout_e]

  idx2 = nbr_edges.reshape(-1)
  n2_pad, per_w2 = _pad_len(idx2.shape[0])
  idx2 = jnp.concatenate(
      [idx2, jnp.zeros((n2_pad - idx2.shape[0],), jnp.int32)])
  eg = _sc_gather(n2_pad, per_w2, 128)(idx2, etab)

  vfeat_new = _agg_call(vfeat, eg.reshape(-1, D2 * 128), _agg_weights(params),
                        n_nodes)
  return (vfeat_new, efeat_new)
